# baseline (speedup 1.0000x reference)
import jax
import jax.numpy as jnp
from jax.experimental import pallas as pl
from jax.experimental.pallas import tpu as pltpu

_CHUNK = 1024
_NBUF = 6
_PREF = 4


def _body(x_hbm, o_hbm, *refs):
    bufs = refs[:_NBUF]
    isems = refs[_NBUF:2 * _NBUF]
    osems = refs[2 * _NBUF:]
    nch = x_hbm.shape[0] // _CHUNK

    def in_cp(i):
        b = i % _NBUF
        return pltpu.make_async_copy(
            x_hbm.at[pl.ds(i * _CHUNK, _CHUNK)], bufs[b], isems[b]
        )

    def out_cp(i):
        b = i % _NBUF
        return pltpu.make_async_copy(
            bufs[b], o_hbm.at[pl.ds(i * _CHUNK, _CHUNK)], osems[b]
        )

    for j in range(_PREF):
        in_cp(j).start()
    waited = 0
    for i in range(nch):
        in_cp(i).wait()
        out_cp(i).start()
        nxt = i + _PREF
        if nxt < nch:
            k = nxt - _NBUF
            if k >= 0:
                out_cp(k).wait()
                waited = k + 1
            in_cp(nxt).start()
    for k in range(waited, nch):
        out_cp(k).wait()


def kernel(x, question_mask, table_mask, column_mask):
    n, d = x.shape
    scratch = (
        [pltpu.VMEM((_CHUNK, d), x.dtype) for _ in range(_NBUF)]
        + [pltpu.SemaphoreType.DMA] * (2 * _NBUF)
    )
    return pl.pallas_call(
        _body,
        in_specs=[pl.BlockSpec(memory_space=pl.ANY)],
        out_specs=pl.BlockSpec(memory_space=pl.ANY),
        out_shape=jax.ShapeDtypeStruct((n, d), x.dtype),
        scratch_shapes=scratch,
    )(x)


# speedup vs baseline: 1.0776x; 1.0776x over previous
import jax
import jax.numpy as jnp
from jax.experimental import pallas as pl


def _copy_body(x_ref, o_ref):
    o_ref[...] = x_ref[...]


def kernel(x, question_mask, table_mask, column_mask):
    n, d = x.shape
    blk = 7168
    return pl.pallas_call(
        _copy_body,
        grid=(pl.cdiv(n, blk),),
        in_specs=[pl.BlockSpec((blk, d), lambda i: (i, 0))],
        out_specs=pl.BlockSpec((blk, d), lambda i: (i, 0)),
        out_shape=jax.ShapeDtypeStruct((n, d), x.dtype),
    )(x)
